# Initial kernel scaffold; baseline (speedup 1.0000x reference)
import jax
import jax.numpy as jnp
from jax.experimental import pallas as pl


def kernel(inst_sizes, inst_pos, inst_pin_weights):
    raise NotImplementedError("write your pallas kernel here")


# speedup vs baseline: 1.1722x; 1.1722x over previous
import functools

import jax
import jax.numpy as jnp
from jax import lax
from jax.experimental import pallas as pl
from jax.experimental.pallas import tpu as pltpu
from jax.experimental.pallas import tpu_sc as plsc

_N = 100000
_NB = 256
_BS = 1.0 / _NB
_INV_BS = float(_NB)
_MIN_SIZE = _BS * 1.4142135
_SCALE = 1.0 / (_BS * _BS * 100.0)
_NW = 32
_CHUNK = 3136
_NPAD = _NW * _CHUNK
_GROUPS = _CHUNK // 16
_NBINS = _NB * _NB
_KMAX = 7


def _sc_body(x_hbm, y_hbm, sx_hbm, sy_hbm, w_hbm, out_hbm,
             xv, yv, sxv, syv, wv, acc):
    wid = lax.axis_index("s") * 2 + lax.axis_index("c")
    base = wid * _CHUNK
    pltpu.sync_copy(x_hbm.at[pl.ds(base, _CHUNK)], xv)
    pltpu.sync_copy(y_hbm.at[pl.ds(base, _CHUNK)], yv)
    pltpu.sync_copy(sx_hbm.at[pl.ds(base, _CHUNK)], sxv)
    pltpu.sync_copy(sy_hbm.at[pl.ds(base, _CHUNK)], syv)
    pltpu.sync_copy(w_hbm.at[pl.ds(base, _CHUNK)], wv)

    zero16 = jnp.zeros((16,), jnp.float32)

    def zero_body(i, c):
        acc[pl.ds(i * 16, 16)] = zero16
        return c

    lax.fori_loop(0, _NBINS // 16, zero_body, 0)

    def group_body(g, c):
        s = g * 16
        x = xv[pl.ds(s, 16)]
        y = yv[pl.ds(s, 16)]
        sx = jnp.maximum(sxv[pl.ds(s, 16)], _MIN_SIZE)
        sy = jnp.maximum(syv[pl.ds(s, 16)], _MIN_SIZE)
        w = wv[pl.ds(s, 16)]
        hx = 0.5 * sx
        hy = 0.5 * sy
        x_min = x - hx
        x_max = x + hx
        y_min = y - hy
        y_max = y + hy
        dens = (w * _SCALE) / (sx * sy)
        ix0 = (x_min * _INV_BS + 1024.0).astype(jnp.int32) - 1024
        iy0 = (y_min * _INV_BS + 1024.0).astype(jnp.int32) - 1024

        rowbase = []
        px = []
        mx = []
        for dx in range(_KMAX):
            bx = ix0 + dx
            lo = bx.astype(jnp.float32) * _BS
            ox = jnp.maximum(
                jnp.minimum(x_max, lo + _BS) - jnp.maximum(x_min, lo), 0.0)
            px.append(dens * ox)
            mx.append((bx >= 0) & (bx < _NB))
            rowbase.append(bx * _NB)

        col = []
        py = []
        my = []
        for dy in range(_KMAX):
            by = iy0 + dy
            lo = by.astype(jnp.float32) * _BS
            oy = jnp.maximum(
                jnp.minimum(y_max, lo + _BS) - jnp.maximum(y_min, lo), 0.0)
            py.append(oy)
            my.append((by >= 0) & (by < _NB))
            col.append(by)

        for dx in range(_KMAX):
            for dy in range(_KMAX):
                idx = rowbase[dx] + col[dy]
                val = px[dx] * py[dy]
                m = mx[dx] & my[dy]
                plsc.addupdate_scatter(acc, [idx], val, mask=m)
        return c

    lax.fori_loop(0, _GROUPS, group_body, 0)

    pltpu.sync_copy(acc, out_hbm.at[wid])


@jax.jit
def _sc_maps(x, y, sx, sy, w):
    mesh = plsc.VectorSubcoreMesh(core_axis_name="c", subcore_axis_name="s")
    return pl.kernel(
        _sc_body,
        out_type=jax.ShapeDtypeStruct((_NW, _NBINS), jnp.float32),
        mesh=mesh,
        compiler_params=pltpu.CompilerParams(needs_layout_passes=False),
        scratch_types=[
            pltpu.VMEM((_CHUNK,), jnp.float32),
            pltpu.VMEM((_CHUNK,), jnp.float32),
            pltpu.VMEM((_CHUNK,), jnp.float32),
            pltpu.VMEM((_CHUNK,), jnp.float32),
            pltpu.VMEM((_CHUNK,), jnp.float32),
            pltpu.VMEM((_NBINS,), jnp.float32),
        ],
    )(x, y, sx, sy, w)


def _reduce_body(maps_ref, out_ref):
    out_ref[...] = jnp.sum(maps_ref[...], axis=0)


@jax.jit
def _reduce(maps):
    return pl.pallas_call(
        _reduce_body,
        out_shape=jax.ShapeDtypeStruct((_NB, _NB), jnp.float32),
    )(maps.reshape(_NW, _NB, _NB))


def kernel(inst_sizes, inst_pos, inst_pin_weights):
    pad = _NPAD - _N
    x = jnp.concatenate([inst_pos[:, 0], jnp.full((pad,), 0.5, jnp.float32)])
    y = jnp.concatenate([inst_pos[:, 1], jnp.full((pad,), 0.5, jnp.float32)])
    sx = jnp.concatenate([inst_sizes[:, 0], jnp.ones((pad,), jnp.float32)])
    sy = jnp.concatenate([inst_sizes[:, 1], jnp.ones((pad,), jnp.float32)])
    w = jnp.concatenate([inst_pin_weights, jnp.zeros((pad,), jnp.float32)])
    maps = _sc_maps(x, y, sx, sy, w)
    return _reduce(maps)
